# baseline (speedup 1.0000x reference)
import functools
import math

import jax
import jax.numpy as jnp
from jax.experimental import pallas as pl
from jax.experimental.pallas import tpu as pltpu


def _rope_block(pe_ref, out_ref, sin_tab, cos_tab, bsin_tab, bcos_tab, *,
                block_rows, d_model, table_rows, n_base):
    d_half = d_model // 2
    log_scale = jnp.float32(-2.0 * math.log(10000.0) / d_model)
    i = pl.program_id(0)

    @pl.when(i == 0)
    def _init_tables():
        fine = 16

        def _sincos(n, step):
            r = jax.lax.broadcasted_iota(
                jnp.int32, (n, d_half), 0).astype(jnp.float32) * step
            c = jax.lax.broadcasted_iota(
                jnp.int32, (n, d_half), 1).astype(jnp.float32)
            a = r * jnp.exp(c * log_scale)
            return jnp.sin(a), jnp.cos(a)

        fs, fc = _sincos(fine, 1)
        cs, cc = _sincos(table_rows // fine, fine)
        for q in range(table_rows // fine):
            sq = cs[q:q + 1, :]
            cq = cc[q:q + 1, :]
            tr = pl.ds(q * fine, fine)
            sin_tab[tr, :] = sq * fc + cq * fs
            cos_tab[tr, :] = cq * fc - sq * fs
        bfs, bfc = _sincos(fine, table_rows)
        bcs, bcc = _sincos(n_base // fine, table_rows * fine)
        for q in range(n_base // fine):
            sq = bcs[q:q + 1, :]
            cq = bcc[q:q + 1, :]
            tr = pl.ds(q * fine, fine)
            bsin_tab[tr, :] = sq * bfc + cq * bfs
            bcos_tab[tr, :] = cq * bfc - sq * bfs

    sr = sin_tab[...]
    cr = cos_tab[...]
    t2 = sr - cr
    t3 = sr + cr
    subs = block_rows // table_rows
    for u in range(subs):
        bi = i * subs + u
        sb = bsin_tab[pl.ds(bi, 1), :]
        cb = bcos_tab[pl.ds(bi, 1), :]
        rs = pl.ds(u * table_rows, table_rows)
        m1 = (sb + cb) * cr
        out_ref[rs, :d_half] = pe_ref[rs, :d_half] + (m1 + cb * t2)
        out_ref[rs, d_half:] = pe_ref[rs, d_half:] + (m1 - sb * t3)


def kernel(x, pe):
    seq_len = x.shape[1]
    d_model = pe.shape[1]
    block_rows = 2048
    table_rows = 128
    grid = seq_len // block_rows
    n_base = seq_len // table_rows
    return pl.pallas_call(
        functools.partial(_rope_block, block_rows=block_rows, d_model=d_model,
                          table_rows=table_rows, n_base=n_base),
        grid=(grid,),
        in_specs=[pl.BlockSpec((block_rows, d_model), lambda i: (i, 0))],
        out_specs=pl.BlockSpec((block_rows, d_model), lambda i: (i, 0)),
        out_shape=jax.ShapeDtypeStruct((seq_len, d_model), jnp.float32),
        scratch_shapes=[
            pltpu.VMEM((table_rows, d_model // 2), jnp.float32),
            pltpu.VMEM((table_rows, d_model // 2), jnp.float32),
            pltpu.VMEM((n_base, d_model // 2), jnp.float32),
            pltpu.VMEM((n_base, d_model // 2), jnp.float32),
        ],
    )(pe)


# speedup vs baseline: 1.0091x; 1.0091x over previous
import functools
import math

import jax
import jax.numpy as jnp
from jax.experimental import pallas as pl
from jax.experimental.pallas import tpu as pltpu


def _rope_block(pe_ref, out_ref, sin_tab, cos_tab, bsin_tab, bcos_tab, *,
                block_rows, d_model, table_rows, n_base):
    d_half = d_model // 2
    log_scale = jnp.float32(-2.0 * math.log(10000.0) / d_model)
    i = pl.program_id(0)

    @pl.when(i == 0)
    def _init_tables():
        fine = 16

        def _sincos(n, step):
            r = jax.lax.broadcasted_iota(
                jnp.int32, (n, d_half), 0).astype(jnp.float32) * step
            c = jax.lax.broadcasted_iota(
                jnp.int32, (n, d_half), 1).astype(jnp.float32)
            a = r * jnp.exp(c * log_scale)
            return jnp.sin(a), jnp.cos(a)

        fs, fc = _sincos(fine, 1)
        cs, cc = _sincos(table_rows // fine, fine)
        for q in range(table_rows // fine):
            sq = cs[q:q + 1, :]
            cq = cc[q:q + 1, :]
            tr = pl.ds(q * fine, fine)
            sin_tab[tr, :] = sq * fc + cq * fs
            cos_tab[tr, :] = cq * fc - sq * fs
        bfs, bfc = _sincos(fine, table_rows)
        bcs, bcc = _sincos(n_base // fine, table_rows * fine)
        for q in range(n_base // fine):
            sq = bcs[q:q + 1, :]
            cq = bcc[q:q + 1, :]
            tr = pl.ds(q * fine, fine)
            bsin_tab[tr, :] = sq * bfc + cq * bfs
            bcos_tab[tr, :] = cq * bfc - sq * bfs

    sr = sin_tab[...]
    cr = cos_tab[...]
    subs = block_rows // table_rows
    for u in range(subs):
        bi = i * subs + u
        sb = bsin_tab[pl.ds(bi, 1), :]
        cb = bcos_tab[pl.ds(bi, 1), :]
        rs = pl.ds(u * table_rows, table_rows)
        out_ref[rs, :d_half] = pe_ref[rs, :d_half] + (sb * cr + cb * sr)
        out_ref[rs, d_half:] = pe_ref[rs, d_half:] + (cb * cr - sb * sr)


def kernel(x, pe):
    seq_len = x.shape[1]
    d_model = pe.shape[1]
    block_rows = 2048
    table_rows = 256
    grid = seq_len // block_rows
    n_base = seq_len // table_rows
    return pl.pallas_call(
        functools.partial(_rope_block, block_rows=block_rows, d_model=d_model,
                          table_rows=table_rows, n_base=n_base),
        grid=(grid,),
        in_specs=[pl.BlockSpec((block_rows, d_model), lambda i: (i, 0))],
        out_specs=pl.BlockSpec((block_rows, d_model), lambda i: (i, 0)),
        out_shape=jax.ShapeDtypeStruct((seq_len, d_model), jnp.float32),
        scratch_shapes=[
            pltpu.VMEM((table_rows, d_model // 2), jnp.float32),
            pltpu.VMEM((table_rows, d_model // 2), jnp.float32),
            pltpu.VMEM((n_base, d_model // 2), jnp.float32),
            pltpu.VMEM((n_base, d_model // 2), jnp.float32),
        ],
    )(pe)
